# baseline (speedup 1.0000x reference)
import jax
import jax.numpy as jnp
from jax import lax
from jax.experimental import pallas as pl

_ALPHA = 0.5
_B_BLOCK = 2


def _linear_kernel(x_ref, w_ref, b_ref, o_ref):
    w = _ALPHA * w_ref[0] + (1.0 - _ALPHA) * w_ref[1]
    bcol = (_ALPHA * b_ref[0] + (1.0 - _ALPHA) * b_ref[1])[:, None]
    for bb in range(_B_BLOCK):
        acc = lax.dot_general(
            w, x_ref[bb],
            dimension_numbers=(((1,), (1,)), ((), ())),
            preferred_element_type=jnp.float32,
        )
        o_ref[bb] = acc + bcol


def kernel(x, At, W_in, b_in, W_out, b_out):
    del At
    Bd, Nd, L = x.shape
    out_ch = W_in.shape[0]
    w_stack = jnp.stack([W_in, W_out])
    b_stack = jnp.stack([b_in, b_out])

    grid = (Bd // _B_BLOCK,)
    out_t = pl.pallas_call(
        _linear_kernel,
        grid=grid,
        in_specs=[
            pl.BlockSpec((_B_BLOCK, Nd, L), lambda i: (i, 0, 0)),
            pl.BlockSpec((2, out_ch, L), lambda i: (0, 0, 0)),
            pl.BlockSpec((2, out_ch), lambda i: (0, 0)),
        ],
        out_specs=pl.BlockSpec((_B_BLOCK, out_ch, Nd), lambda i: (i, 0, 0)),
        out_shape=jax.ShapeDtypeStruct((Bd, out_ch, Nd), jnp.float32),
    )(x, w_stack, b_stack)
    return out_t.transpose(0, 2, 1)


# speedup vs baseline: 1.4790x; 1.4790x over previous
import jax
import jax.numpy as jnp
from jax import lax
from jax.experimental import pallas as pl
from jax.experimental.pallas import tpu as pltpu

_ALPHA = 0.5


def _linear_kernel(x_ref, w_in_ref, b_in_ref, w_out_ref, b_out_ref, o_ref):
    w = _ALPHA * w_in_ref[...] + (1.0 - _ALPHA) * w_out_ref[...]
    bcol = (_ALPHA * b_in_ref[...] + (1.0 - _ALPHA) * b_out_ref[...])[:, None]
    for bb in range(x_ref.shape[0]):
        acc = lax.dot_general(
            w, x_ref[bb],
            dimension_numbers=(((1,), (1,)), ((), ())),
            preferred_element_type=jnp.float32,
        )
        o_ref[bb] = acc + bcol


def kernel(x, At, W_in, b_in, W_out, b_out):
    del At
    Bd, Nd, L = x.shape
    out_ch = W_in.shape[0]

    vmem = pltpu.MemorySpace.VMEM
    out_t = pl.pallas_call(
        _linear_kernel,
        in_specs=[pl.BlockSpec(memory_space=vmem)] * 5,
        out_specs=pl.BlockSpec(memory_space=vmem),
        out_shape=jax.ShapeDtypeStruct((Bd, out_ch, Nd), jnp.float32),
    )(x, W_in, b_in, W_out, b_out)
    return out_t.transpose(0, 2, 1)


# speedup vs baseline: 1.7170x; 1.1609x over previous
import jax
import jax.numpy as jnp
from jax import lax
from jax.experimental import pallas as pl
from jax.experimental.pallas import tpu as pltpu

_ALPHA = 0.5
_B_BLOCK = 2


def _linear_kernel(x_ref, w_in_ref, b_in_ref, w_out_ref, b_out_ref, o_ref):
    w = _ALPHA * w_in_ref[...] + (1.0 - _ALPHA) * w_out_ref[...]
    b = _ALPHA * b_in_ref[...] + (1.0 - _ALPHA) * b_out_ref[...]
    bcol = b[:, None]
    for bb in range(_B_BLOCK):
        acc = lax.dot_general(
            w, x_ref[bb],
            dimension_numbers=(((1,), (1,)), ((), ())),
            preferred_element_type=jnp.float32,
        )
        o_ref[bb] = acc + bcol


def kernel(x, At, W_in, b_in, W_out, b_out):
    del At
    Bd, Nd, L = x.shape
    out_ch = W_in.shape[0]

    grid = (Bd // _B_BLOCK,)
    out_t = pl.pallas_call(
        _linear_kernel,
        grid=grid,
        in_specs=[
            pl.BlockSpec((_B_BLOCK, Nd, L), lambda i: (i, 0, 0)),
            pl.BlockSpec((out_ch, L), lambda i: (0, 0)),
            pl.BlockSpec((out_ch,), lambda i: (0,)),
            pl.BlockSpec((out_ch, L), lambda i: (0, 0)),
            pl.BlockSpec((out_ch,), lambda i: (0,)),
        ],
        out_specs=pl.BlockSpec((_B_BLOCK, out_ch, Nd), lambda i: (i, 0, 0)),
        out_shape=jax.ShapeDtypeStruct((Bd, out_ch, Nd), jnp.float32),
        compiler_params=pltpu.CompilerParams(
            skip_device_barrier=True,
            disable_bounds_checks=True,
            disable_semaphore_checks=True,
        ),
    )(x, W_in, b_in, W_out, b_out)
    return out_t.transpose(0, 2, 1)
